# baseline (speedup 1.0000x reference)
import functools

import jax
import jax.numpy as jnp
from jax.experimental import pallas as pl


def _moe_block(x_ref, wg_ref, bg_ref, w1_ref, b1_ref, w2_ref, b2_ref, o_ref,
               *, num_experts):
    xb = x_ref[...]
    logits = jnp.dot(xb, wg_ref[...], preferred_element_type=jnp.float32)
    logits = logits + bg_ref[...]
    m = jnp.max(logits, axis=-1, keepdims=True)
    p = jnp.exp(logits - m)
    p = p / jnp.sum(p, axis=-1, keepdims=True)

    i1 = jnp.argmax(p, axis=-1)[:, None]
    top1 = jnp.max(p, axis=-1, keepdims=True)
    cols = jax.lax.broadcasted_iota(jnp.int32, p.shape, 1)
    p2 = jnp.where(cols == i1, -jnp.inf, p)
    i2 = jnp.argmax(p2, axis=-1)[:, None]
    top2 = jnp.max(p2, axis=-1, keepdims=True)
    denom = top1 + top2

    acc = jnp.zeros_like(xb)
    xb16 = xb.astype(jnp.bfloat16)
    for e in range(num_experts):
        w_e = (jnp.where(i1 == e, top1, 0.0) + jnp.where(i2 == e, top2, 0.0)) / denom
        h = jnp.dot(xb16, w1_ref[e].astype(jnp.bfloat16),
                    preferred_element_type=jnp.float32) + b1_ref[e]
        h = jnp.maximum(h, 0.0)
        y = jnp.dot(h.astype(jnp.bfloat16), w2_ref[e].astype(jnp.bfloat16),
                    preferred_element_type=jnp.float32) + b2_ref[e]
        acc = acc + w_e * y
    o_ref[...] = acc


def kernel(x, Wg, bg, W1, b1, W2, b2):
    B, S, H = x.shape
    E, _, F = W1.shape
    N = B * S
    xf = x.reshape(N, H)
    M = 2048
    grid = (N // M,)

    out = pl.pallas_call(
        functools.partial(_moe_block, num_experts=E),
        grid=grid,
        in_specs=[
            pl.BlockSpec((M, H), lambda i: (i, 0)),
            pl.BlockSpec((H, E), lambda i: (0, 0)),
            pl.BlockSpec((1, E), lambda i: (0, 0)),
            pl.BlockSpec((E, H, F), lambda i: (0, 0, 0)),
            pl.BlockSpec((E, 1, F), lambda i: (0, 0, 0)),
            pl.BlockSpec((E, F, H), lambda i: (0, 0, 0)),
            pl.BlockSpec((E, 1, H), lambda i: (0, 0, 0)),
        ],
        out_specs=pl.BlockSpec((M, H), lambda i: (i, 0)),
        out_shape=jax.ShapeDtypeStruct((N, H), jnp.float32),
    )(xf, Wg, bg.reshape(1, E), W1, b1.reshape(E, 1, F), W2, b2.reshape(E, 1, H))
    return out.reshape(B, S, H)


# speedup vs baseline: 1.0017x; 1.0017x over previous
import functools

import jax
import jax.numpy as jnp
from jax.experimental import pallas as pl


def _moe_block(x_ref, wg_ref, bg_ref, w1_ref, b1_ref, w2_ref, b2_ref, o_ref,
               *, num_experts):
    xb = x_ref[...]
    logits = jnp.dot(xb, wg_ref[...], preferred_element_type=jnp.float32)
    logits = logits + bg_ref[...]
    m = jnp.max(logits, axis=-1, keepdims=True)
    p = jnp.exp(logits - m)
    p = p / jnp.sum(p, axis=-1, keepdims=True)

    i1 = jnp.argmax(p, axis=-1)[:, None]
    top1 = jnp.max(p, axis=-1, keepdims=True)
    cols = jax.lax.broadcasted_iota(jnp.int32, p.shape, 1)
    p2 = jnp.where(cols == i1, -jnp.inf, p)
    i2 = jnp.argmax(p2, axis=-1)[:, None]
    top2 = jnp.max(p2, axis=-1, keepdims=True)
    denom = top1 + top2

    acc = jnp.zeros_like(xb)
    xb16 = xb.astype(jnp.bfloat16)
    for e in range(num_experts):
        w_e = (jnp.where(i1 == e, top1, 0.0) + jnp.where(i2 == e, top2, 0.0)) / denom
        h = jnp.dot(xb16, w1_ref[e].astype(jnp.bfloat16),
                    preferred_element_type=jnp.float32) + b1_ref[e]
        h = jnp.maximum(h, 0.0)
        y = jnp.dot(h.astype(jnp.bfloat16), w2_ref[e].astype(jnp.bfloat16),
                    preferred_element_type=jnp.float32) + b2_ref[e]
        acc = acc + w_e * y
    o_ref[...] = acc


def kernel(x, Wg, bg, W1, b1, W2, b2):
    B, S, H = x.shape
    E, _, F = W1.shape
    N = B * S
    xf = x.reshape(N, H)
    M = 1024
    grid = (N // M,)

    out = pl.pallas_call(
        functools.partial(_moe_block, num_experts=E),
        grid=grid,
        in_specs=[
            pl.BlockSpec((M, H), lambda i: (i, 0)),
            pl.BlockSpec((H, E), lambda i: (0, 0)),
            pl.BlockSpec((1, E), lambda i: (0, 0)),
            pl.BlockSpec((E, H, F), lambda i: (0, 0, 0)),
            pl.BlockSpec((E, 1, F), lambda i: (0, 0, 0)),
            pl.BlockSpec((E, F, H), lambda i: (0, 0, 0)),
            pl.BlockSpec((E, 1, H), lambda i: (0, 0, 0)),
        ],
        out_specs=pl.BlockSpec((M, H), lambda i: (i, 0)),
        out_shape=jax.ShapeDtypeStruct((N, H), jnp.float32),
    )(xf, Wg, bg.reshape(1, E), W1, b1.reshape(E, 1, F), W2, b2.reshape(E, 1, H))
    return out.reshape(B, S, H)
